# baseline (speedup 1.0000x reference)
import functools

import jax
import jax.numpy as jnp
from jax import lax
from jax.experimental import pallas as pl
from jax.experimental.pallas import tpu as pltpu
from jax.experimental.pallas import tpu_sc as plsc

_K = 20
_NAGG = 100
_ITERS = 10


def _topk_body(a0_ref, a1_ref, out_ref, idx_scr):
    L, N, B = a0_ref.shape
    x0 = a0_ref[0]
    x1 = a1_ref[0]
    for l in range(1, L):
        x0 = x0 + a0_ref[l]
        x1 = x1 + a1_ref[l]
    x0 = x0 * (1.0 / L)
    x1 = x1 * (1.0 / L)
    m = jnp.maximum(x0, x1)
    e0 = jnp.exp(x0 - m)
    e1 = jnp.exp(x1 - m)
    p = e1 / (e0 + e1)

    iota_n = lax.broadcasted_iota(jnp.int32, (N, B), 0)

    def body(j, s):
        mx = jnp.max(s, axis=0, keepdims=True)
        sel = jnp.min(jnp.where(s == mx, iota_n, N), axis=0, keepdims=True)
        idx_scr[pl.ds(j, 1), :] = sel
        return jnp.where(iota_n == sel, -1.0, s)

    lax.fori_loop(0, _NAGG, body, p)

    idx_all = jnp.transpose(idx_scr[...])
    l_iota = lax.broadcasted_iota(jnp.int32, (B, _NAGG, L), 2)
    b_iota = lax.broadcasted_iota(jnp.int32, (B, _NAGG, L), 0)
    out_ref[...] = l_iota * (B * N) + b_iota * N + idx_all[:, :, None]


def _topk_call(a0, a1):
    L, N, B = a0.shape
    return pl.pallas_call(
        _topk_body,
        out_shape=jax.ShapeDtypeStruct((B, _NAGG, L), jnp.int32),
        scratch_shapes=[pltpu.VMEM((_NAGG, B), jnp.int32)],
    )(a0, a1)


_CHUNK = 40


def _gather_body(table_hbm, idx_hbm, out_hbm, idx_v, buf0, buf1, sem0, sem1):
    rows_total = idx_hbm.shape[0]
    nw = 32
    per_w = rows_total // nw
    nchunk = per_w // _CHUNK
    wid = lax.axis_index("s") * 2 + lax.axis_index("c")
    base = wid * per_w
    pltpu.sync_copy(idx_hbm.at[pl.ds(base, per_w)], idx_v)
    bufs = (buf0, buf1)
    sems = (sem0, sem1)

    def start(c, slot):
        return pltpu.async_copy(
            table_hbm.at[idx_v.at[pl.ds(c * _CHUNK, _CHUNK)]], bufs[slot], sems[slot])

    cp = start(0, 0)
    for c in range(nchunk):
        nxt = start(c + 1, (c + 1) % 2) if c + 1 < nchunk else None
        cp.wait()
        pltpu.sync_copy(bufs[c % 2], out_hbm.at[pl.ds(base + c * _CHUNK, _CHUNK)])
        cp = nxt


def _gather_call(table, idx_flat):
    rows, d = idx_flat.shape[0], table.shape[1]
    mesh = plsc.VectorSubcoreMesh(core_axis_name="c", subcore_axis_name="s")
    k = functools.partial(
        pl.kernel,
        mesh=mesh,
        out_type=jax.ShapeDtypeStruct((rows, d), jnp.float32),
        scratch_types=[
            pltpu.VMEM((rows // 32,), jnp.int32),
            pltpu.VMEM((_CHUNK, d), jnp.float32),
            pltpu.VMEM((_CHUNK, d), jnp.float32),
            pltpu.SemaphoreType.DMA,
            pltpu.SemaphoreType.DMA,
        ],
    )(_gather_body)
    return k(table, idx_flat)


def _km_body(sel_ref, out_ref, *, L, D):
    BN = sel_ref.shape[0]
    B = BN // _NAGG
    NA, LD = _NAGG, L * D
    BK = B * _K
    x = sel_ref[...]

    row_cl = lax.broadcasted_iota(jnp.int32, (BK, BN), 0)
    col_b = lax.broadcasted_iota(jnp.int32, (BK, BN), 1) // _NAGG
    penalty = jnp.where(row_cl // _K == col_b, 0.0, 1e9)

    def labels_onehot(c):
        c2 = jnp.sum(c * c, axis=1, keepdims=True)
        g = lax.dot_general(c, x, (((1,), (1,)), ((), ())),
                            preferred_element_type=jnp.float32,
                            precision=lax.Precision.HIGHEST)
        d2 = c2 - 2.0 * g + penalty
        mn = jnp.min(d2, axis=0, keepdims=True)
        lbl = jnp.min(jnp.where(d2 == mn, row_cl, BK), axis=0, keepdims=True)
        return (row_cl == lbl).astype(jnp.float32)

    def body(i, c):
        onehot = labels_onehot(c)
        counts = jnp.sum(onehot, axis=1, keepdims=True)
        s = lax.dot_general(onehot, x, (((1,), (0,)), ((), ())),
                            preferred_element_type=jnp.float32,
                            precision=lax.Precision.HIGHEST)
        return jnp.where(counts > 0.0, s / jnp.maximum(counts, 1.0), c)

    c0 = jnp.concatenate([x[b * NA:b * NA + _K, :] for b in range(B)], axis=0)
    c = lax.fori_loop(0, _ITERS, body, c0)

    onehot = labels_onehot(c)
    counts = jnp.sum(onehot, axis=1, keepdims=True)
    sum_x = x[:, 0:D]
    for l in range(1, L):
        sum_x = sum_x + x[:, l * D:(l + 1) * D]
    pooled = lax.dot_general(onehot, sum_x, (((1,), (0,)), ((), ())),
                             preferred_element_type=jnp.float32,
                             precision=lax.Precision.HIGHEST)
    centers = pooled / jnp.maximum(L * counts, 1.0)
    sel_b = (lax.broadcasted_iota(jnp.int32, (B, BK), 1) // _K
             == lax.broadcasted_iota(jnp.int32, (B, BK), 0)).astype(jnp.float32)
    ob = lax.dot_general(sel_b, centers, (((1,), (0,)), ((), ())),
                         preferred_element_type=jnp.float32,
                         precision=lax.Precision.HIGHEST) * (1.0 / _K)
    nrm = jnp.sqrt(jnp.sum(ob * ob, axis=1, keepdims=True))
    out_ref[...] = ob / jnp.maximum(nrm, 1e-12)


_KM_PROGS = 2


def _km_call(sel, L, D):
    BN, LD = sel.shape
    B = BN // _NAGG
    p = _KM_PROGS
    return pl.pallas_call(
        functools.partial(_km_body, L=L, D=D),
        grid=(p,),
        in_specs=[pl.BlockSpec((BN // p, LD), lambda i: (i, 0))],
        out_specs=pl.BlockSpec((B // p, D), lambda i: (i, 0)),
        out_shape=jax.ShapeDtypeStruct((B, D), jnp.float32),
        compiler_params=pltpu.CompilerParams(
            vmem_limit_bytes=110 * 1024 * 1024),
    )(sel)


def kernel(patch_tokens, anomaly_maps):
    L, B, N, D = patch_tokens.shape
    a0 = anomaly_maps[..., 0].transpose(0, 2, 1)
    a1 = anomaly_maps[..., 1].transpose(0, 2, 1)
    idx_flat = _topk_call(a0, a1)
    table = patch_tokens.reshape(L * B * N, D)
    rows = _gather_call(table, idx_flat.reshape(B * _NAGG * L))
    sel = rows.reshape(B * _NAGG, L * D)
    return _km_call(sel, L, D)


# speedup vs baseline: 2.4960x; 2.4960x over previous
import functools

import jax
import jax.numpy as jnp
from jax import lax
from jax.experimental import pallas as pl
from jax.experimental.pallas import tpu as pltpu
from jax.experimental.pallas import tpu_sc as plsc

_K = 20
_NAGG = 100
_ITERS = 10


def _topk_body(a0_ref, a1_ref, out_ref, idx_scr):
    L, N, B = a0_ref.shape
    x0 = a0_ref[0]
    x1 = a1_ref[0]
    for l in range(1, L):
        x0 = x0 + a0_ref[l]
        x1 = x1 + a1_ref[l]
    x0 = x0 * (1.0 / L)
    x1 = x1 * (1.0 / L)
    m = jnp.maximum(x0, x1)
    e0 = jnp.exp(x0 - m)
    e1 = jnp.exp(x1 - m)
    p = e1 / (e0 + e1)

    iota_n = lax.broadcasted_iota(jnp.int32, (N, B), 0)

    def body(j, s):
        mx = jnp.max(s, axis=0, keepdims=True)
        sel = jnp.min(jnp.where(s == mx, iota_n, N), axis=0, keepdims=True)
        idx_scr[pl.ds(j, 1), :] = sel
        return jnp.where(iota_n == sel, -1.0, s)

    lax.fori_loop(0, _NAGG, body, p)

    idx_all = jnp.transpose(idx_scr[...])
    l_iota = lax.broadcasted_iota(jnp.int32, (B, _NAGG, L), 2)
    b_iota = lax.broadcasted_iota(jnp.int32, (B, _NAGG, L), 0)
    out_ref[...] = l_iota * (B * N) + b_iota * N + idx_all[:, :, None]


def _topk_call(a0, a1):
    L, N, B = a0.shape
    return pl.pallas_call(
        _topk_body,
        out_shape=jax.ShapeDtypeStruct((B, _NAGG, L), jnp.int32),
        scratch_shapes=[pltpu.VMEM((_NAGG, B), jnp.int32)],
    )(a0, a1)


_CHUNK = 40


def _gather_body(table_hbm, idx_hbm, out_hbm, idx_v, buf0, buf1, sem0, sem1):
    rows_total = idx_hbm.shape[0]
    nw = 32
    per_w = rows_total // nw
    nchunk = per_w // _CHUNK
    wid = lax.axis_index("s") * 2 + lax.axis_index("c")
    base = wid * per_w
    pltpu.sync_copy(idx_hbm.at[pl.ds(base, per_w)], idx_v)
    bufs = (buf0, buf1)
    sems = (sem0, sem1)

    def start(c, slot):
        return pltpu.async_copy(
            table_hbm.at[idx_v.at[pl.ds(c * _CHUNK, _CHUNK)]], bufs[slot], sems[slot])

    cp = start(0, 0)
    for c in range(nchunk):
        nxt = start(c + 1, (c + 1) % 2) if c + 1 < nchunk else None
        cp.wait()
        pltpu.sync_copy(bufs[c % 2], out_hbm.at[pl.ds(base + c * _CHUNK, _CHUNK)])
        cp = nxt


def _gather_call(table, idx_flat):
    rows, d = idx_flat.shape[0], table.shape[1]
    mesh = plsc.VectorSubcoreMesh(core_axis_name="c", subcore_axis_name="s")
    k = functools.partial(
        pl.kernel,
        mesh=mesh,
        out_type=jax.ShapeDtypeStruct((rows, d), jnp.float32),
        scratch_types=[
            pltpu.VMEM((rows // 32,), jnp.int32),
            pltpu.VMEM((_CHUNK, d), jnp.float32),
            pltpu.VMEM((_CHUNK, d), jnp.float32),
            pltpu.SemaphoreType.DMA,
            pltpu.SemaphoreType.DMA,
        ],
    )(_gather_body)
    return k(table, idx_flat)


def _km_body(sel_ref, out_ref, *, L, D):
    BN = sel_ref.shape[0]
    B = BN // _NAGG
    NA, LD = _NAGG, L * D
    BK = B * _K
    x = sel_ref[...]

    row_cl = lax.broadcasted_iota(jnp.int32, (BK, BN), 0)
    col_b = lax.broadcasted_iota(jnp.int32, (BK, BN), 1) // _NAGG
    penalty = jnp.where(row_cl // _K == col_b, 0.0, 1e9)

    def labels_lbl(c):
        c2 = jnp.sum(c * c, axis=1, keepdims=True)
        g = lax.dot_general(c, x, (((1,), (1,)), ((), ())),
                            preferred_element_type=jnp.float32,
                            precision=lax.Precision.HIGHEST)
        d2 = c2 - 2.0 * g + penalty
        mn = jnp.min(d2, axis=0, keepdims=True)
        return jnp.min(jnp.where(d2 == mn, row_cl, BK), axis=0, keepdims=True)

    def labels_onehot(c):
        return (row_cl == labels_lbl(c)).astype(jnp.float32)

    def cond(carry):
        i, c, lbl_prev, changed = carry
        return jnp.logical_and(i < _ITERS, changed)

    def body(carry):
        i, c, lbl_prev, _ = carry
        lbl = labels_lbl(c)
        changed = jnp.any(lbl != lbl_prev)
        onehot = (row_cl == lbl).astype(jnp.float32)
        counts = jnp.sum(onehot, axis=1, keepdims=True)
        s = lax.dot_general(onehot, x, (((1,), (0,)), ((), ())),
                            preferred_element_type=jnp.float32,
                            precision=lax.Precision.HIGHEST)
        c = jnp.where(counts > 0.0, s / jnp.maximum(counts, 1.0), c)
        return (i + 1, c, lbl, changed)

    c0 = jnp.concatenate([x[b * NA:b * NA + _K, :] for b in range(B)], axis=0)
    lbl0 = jnp.full((1, BN), -1, dtype=jnp.int32)
    _, c, _, _ = lax.while_loop(cond, body, (0, c0, lbl0, True))

    onehot = labels_onehot(c)
    counts = jnp.sum(onehot, axis=1, keepdims=True)
    sum_x = x[:, 0:D]
    for l in range(1, L):
        sum_x = sum_x + x[:, l * D:(l + 1) * D]
    pooled = lax.dot_general(onehot, sum_x, (((1,), (0,)), ((), ())),
                             preferred_element_type=jnp.float32,
                             precision=lax.Precision.HIGHEST)
    centers = pooled / jnp.maximum(L * counts, 1.0)
    sel_b = (lax.broadcasted_iota(jnp.int32, (B, BK), 1) // _K
             == lax.broadcasted_iota(jnp.int32, (B, BK), 0)).astype(jnp.float32)
    ob = lax.dot_general(sel_b, centers, (((1,), (0,)), ((), ())),
                         preferred_element_type=jnp.float32,
                         precision=lax.Precision.HIGHEST) * (1.0 / _K)
    nrm = jnp.sqrt(jnp.sum(ob * ob, axis=1, keepdims=True))
    out_ref[...] = ob / jnp.maximum(nrm, 1e-12)


_KM_PROGS = 2


def _km_call(sel, L, D):
    BN, LD = sel.shape
    B = BN // _NAGG
    p = _KM_PROGS
    return pl.pallas_call(
        functools.partial(_km_body, L=L, D=D),
        grid=(p,),
        in_specs=[pl.BlockSpec((BN // p, LD), lambda i: (i, 0))],
        out_specs=pl.BlockSpec((B // p, D), lambda i: (i, 0)),
        out_shape=jax.ShapeDtypeStruct((B, D), jnp.float32),
        compiler_params=pltpu.CompilerParams(
            vmem_limit_bytes=110 * 1024 * 1024),
    )(sel)


def kernel(patch_tokens, anomaly_maps):
    L, B, N, D = patch_tokens.shape
    a0 = anomaly_maps[..., 0].transpose(0, 2, 1)
    a1 = anomaly_maps[..., 1].transpose(0, 2, 1)
    idx_flat = _topk_call(a0, a1)
    table = patch_tokens.reshape(L * B * N, D)
    rows = _gather_call(table, idx_flat.reshape(B * _NAGG * L))
    sel = rows.reshape(B * _NAGG, L * D)
    return _km_call(sel, L, D)


# speedup vs baseline: 3.1975x; 1.2811x over previous
import functools

import jax
import jax.numpy as jnp
from jax import lax
from jax.experimental import pallas as pl
from jax.experimental.pallas import tpu as pltpu
from jax.experimental.pallas import tpu_sc as plsc

_K = 20
_NAGG = 100
_ITERS = 10


def _topk_body(a0_ref, a1_ref, out_ref, idx_scr):
    L, B, N = a0_ref.shape
    x0 = a0_ref[0]
    x1 = a1_ref[0]
    for l in range(1, L):
        x0 = x0 + a0_ref[l]
        x1 = x1 + a1_ref[l]
    x0 = x0 * (1.0 / L)
    x1 = x1 * (1.0 / L)
    m = jnp.maximum(x0, x1)
    e0 = jnp.exp(x0 - m)
    e1 = jnp.exp(x1 - m)
    p = e1 / (e0 + e1)

    iota_n = lax.broadcasted_iota(jnp.int32, (B, N), 1)

    def body(j, s):
        mx = jnp.max(s, axis=1, keepdims=True)
        sel = jnp.min(jnp.where(s == mx, iota_n, N), axis=1, keepdims=True)
        idx_scr[pl.ds(j, 1), :] = jnp.transpose(sel)
        return jnp.where(iota_n == sel, -1.0, s)

    lax.fori_loop(0, _NAGG, body, p)

    idx_all = jnp.transpose(idx_scr[...])
    l_iota = lax.broadcasted_iota(jnp.int32, (B, _NAGG, L), 2)
    b_iota = lax.broadcasted_iota(jnp.int32, (B, _NAGG, L), 0)
    out_ref[...] = l_iota * (B * N) + b_iota * N + idx_all[:, :, None]


def _topk_call(a0, a1):
    L, B, N = a0.shape
    return pl.pallas_call(
        _topk_body,
        out_shape=jax.ShapeDtypeStruct((B, _NAGG, L), jnp.int32),
        scratch_shapes=[pltpu.VMEM((_NAGG, B), jnp.int32)],
    )(a0, a1)


_CHUNK = 40


def _gather_body(table_hbm, idx_hbm, out_hbm, idx_v, buf0, buf1, sem0, sem1):
    rows_total = idx_hbm.shape[0]
    nw = 32
    per_w = rows_total // nw
    nchunk = per_w // _CHUNK
    wid = lax.axis_index("s") * 2 + lax.axis_index("c")
    base = wid * per_w
    pltpu.sync_copy(idx_hbm.at[pl.ds(base, per_w)], idx_v)
    bufs = (buf0, buf1)
    sems = (sem0, sem1)

    def start(c, slot):
        return pltpu.async_copy(
            table_hbm.at[idx_v.at[pl.ds(c * _CHUNK, _CHUNK)]], bufs[slot], sems[slot])

    cp = start(0, 0)
    for c in range(nchunk):
        nxt = start(c + 1, (c + 1) % 2) if c + 1 < nchunk else None
        cp.wait()
        pltpu.sync_copy(bufs[c % 2], out_hbm.at[pl.ds(base + c * _CHUNK, _CHUNK)])
        cp = nxt


def _gather_call(table, idx_flat):
    rows, d = idx_flat.shape[0], table.shape[1]
    mesh = plsc.VectorSubcoreMesh(core_axis_name="c", subcore_axis_name="s")
    k = functools.partial(
        pl.kernel,
        mesh=mesh,
        out_type=jax.ShapeDtypeStruct((rows, d), jnp.float32),
        scratch_types=[
            pltpu.VMEM((rows // 32,), jnp.int32),
            pltpu.VMEM((_CHUNK, d), jnp.float32),
            pltpu.VMEM((_CHUNK, d), jnp.float32),
            pltpu.SemaphoreType.DMA,
            pltpu.SemaphoreType.DMA,
        ],
    )(_gather_body)
    return k(table, idx_flat)


def _km_body(sel_ref, out_ref, *, L, D):
    BN = sel_ref.shape[0]
    B = BN // _NAGG
    NA, LD = _NAGG, L * D
    BK = B * _K
    x = sel_ref[...]
    x_hi = x.astype(jnp.bfloat16)
    x_lo = (x - x_hi.astype(jnp.float32)).astype(jnp.bfloat16)

    row_cl = lax.broadcasted_iota(jnp.int32, (BK, BN), 0)
    col_b = lax.broadcasted_iota(jnp.int32, (BK, BN), 1) // _NAGG
    penalty = jnp.where(row_cl // _K == col_b, 0.0, 1e9)

    def _dotT(a, b):
        return lax.dot_general(a, b, (((1,), (1,)), ((), ())),
                               preferred_element_type=jnp.float32)

    def labels_lbl(c):
        c2 = jnp.sum(c * c, axis=1, keepdims=True)
        c_hi = c.astype(jnp.bfloat16)
        c_lo = (c - c_hi.astype(jnp.float32)).astype(jnp.bfloat16)
        g = _dotT(c_hi, x_hi) + _dotT(c_hi, x_lo) + _dotT(c_lo, x_hi)
        d2 = c2 - 2.0 * g + penalty
        mn = jnp.min(d2, axis=0, keepdims=True)
        return jnp.min(jnp.where(d2 == mn, row_cl, BK), axis=0, keepdims=True)

    def labels_onehot(c):
        return (row_cl == labels_lbl(c)).astype(jnp.float32)

    def cond(carry):
        i, c, lbl_prev, changed = carry
        return jnp.logical_and(i < _ITERS, changed)

    def body(carry):
        i, c, lbl_prev, _ = carry
        lbl = labels_lbl(c)
        changed = jnp.any(lbl != lbl_prev)
        onehot = (row_cl == lbl).astype(jnp.float32)
        counts = jnp.sum(onehot, axis=1, keepdims=True)
        oh16 = onehot.astype(jnp.bfloat16)
        s = (lax.dot_general(oh16, x_hi, (((1,), (0,)), ((), ())),
                             preferred_element_type=jnp.float32)
             + lax.dot_general(oh16, x_lo, (((1,), (0,)), ((), ())),
                               preferred_element_type=jnp.float32))
        c = jnp.where(counts > 0.0, s / jnp.maximum(counts, 1.0), c)
        return (i + 1, c, lbl, changed)

    c0 = jnp.concatenate([x[b * NA:b * NA + _K, :] for b in range(B)], axis=0)
    lbl0 = jnp.full((1, BN), -1, dtype=jnp.int32)
    _, c, _, _ = lax.while_loop(cond, body, (0, c0, lbl0, True))

    onehot = labels_onehot(c)
    counts = jnp.sum(onehot, axis=1, keepdims=True)
    sum_x = x[:, 0:D]
    for l in range(1, L):
        sum_x = sum_x + x[:, l * D:(l + 1) * D]
    pooled = lax.dot_general(onehot, sum_x, (((1,), (0,)), ((), ())),
                             preferred_element_type=jnp.float32,
                             precision=lax.Precision.HIGHEST)
    centers = pooled / jnp.maximum(L * counts, 1.0)
    sel_b = (lax.broadcasted_iota(jnp.int32, (B, BK), 1) // _K
             == lax.broadcasted_iota(jnp.int32, (B, BK), 0)).astype(jnp.float32)
    ob = lax.dot_general(sel_b, centers, (((1,), (0,)), ((), ())),
                         preferred_element_type=jnp.float32,
                         precision=lax.Precision.HIGHEST) * (1.0 / _K)
    nrm = jnp.sqrt(jnp.sum(ob * ob, axis=1, keepdims=True))
    out_ref[...] = ob / jnp.maximum(nrm, 1e-12)


_KM_PROGS = 2


def _km_call(sel, L, D):
    BN, LD = sel.shape
    B = BN // _NAGG
    p = _KM_PROGS
    return pl.pallas_call(
        functools.partial(_km_body, L=L, D=D),
        grid=(p,),
        in_specs=[pl.BlockSpec((BN // p, LD), lambda i: (i, 0))],
        out_specs=pl.BlockSpec((B // p, D), lambda i: (i, 0)),
        out_shape=jax.ShapeDtypeStruct((B, D), jnp.float32),
        compiler_params=pltpu.CompilerParams(
            vmem_limit_bytes=110 * 1024 * 1024),
    )(sel)


def kernel(patch_tokens, anomaly_maps):
    L, B, N, D = patch_tokens.shape
    a0 = anomaly_maps[..., 0]
    a1 = anomaly_maps[..., 1]
    idx_flat = _topk_call(a0, a1)
    table = patch_tokens.reshape(L * B * N, D)
    rows = _gather_call(table, idx_flat.reshape(B * _NAGG * L))
    sel = rows.reshape(B * _NAGG, L * D)
    return _km_call(sel, L, D)


# speedup vs baseline: 3.2510x; 1.0167x over previous
import functools

import jax
import jax.numpy as jnp
from jax import lax
from jax.experimental import pallas as pl
from jax.experimental.pallas import tpu as pltpu
from jax.experimental.pallas import tpu_sc as plsc

_K = 20
_NAGG = 100
_ITERS = 10


def _topk_body(a0_ref, a1_ref, out_ref, idx_scr):
    L, B, N = a0_ref.shape
    x0 = a0_ref[0]
    x1 = a1_ref[0]
    for l in range(1, L):
        x0 = x0 + a0_ref[l]
        x1 = x1 + a1_ref[l]
    x0 = x0 * (1.0 / L)
    x1 = x1 * (1.0 / L)
    m = jnp.maximum(x0, x1)
    e0 = jnp.exp(x0 - m)
    e1 = jnp.exp(x1 - m)
    p = e1 / (e0 + e1)

    iota_n = lax.broadcasted_iota(jnp.int32, (B, N), 1)

    def body(j, s):
        mx = jnp.max(s, axis=1, keepdims=True)
        sel = jnp.min(jnp.where(s == mx, iota_n, N), axis=1, keepdims=True)
        idx_scr[pl.ds(j, 1), :] = jnp.transpose(sel)
        return jnp.where(iota_n == sel, -1.0, s)

    lax.fori_loop(0, _NAGG, body, p)

    idx_all = jnp.transpose(idx_scr[...])
    l_iota = lax.broadcasted_iota(jnp.int32, (B, _NAGG, L), 2)
    b_iota = lax.broadcasted_iota(jnp.int32, (B, _NAGG, L), 0)
    out_ref[...] = l_iota * (B * N) + b_iota * N + idx_all[:, :, None]


def _topk_call(a0, a1):
    L, B, N = a0.shape
    return pl.pallas_call(
        _topk_body,
        out_shape=jax.ShapeDtypeStruct((B, _NAGG, L), jnp.int32),
        scratch_shapes=[pltpu.VMEM((_NAGG, B), jnp.int32)],
    )(a0, a1)


_CHUNK = 40


def _gather_body(table_hbm, idx_hbm, out_hbm, idx_v, buf0, buf1, sem0, sem1):
    rows_total = idx_hbm.shape[0]
    nw = 32
    per_w = rows_total // nw
    nchunk = per_w // _CHUNK
    wid = lax.axis_index("s") * 2 + lax.axis_index("c")
    base = wid * per_w
    pltpu.sync_copy(idx_hbm.at[pl.ds(base, per_w)], idx_v)
    bufs = (buf0, buf1)
    sems = (sem0, sem1)

    def start(c, slot):
        return pltpu.async_copy(
            table_hbm.at[idx_v.at[pl.ds(c * _CHUNK, _CHUNK)]], bufs[slot], sems[slot])

    cp = start(0, 0)
    for c in range(nchunk):
        nxt = start(c + 1, (c + 1) % 2) if c + 1 < nchunk else None
        cp.wait()
        pltpu.sync_copy(bufs[c % 2], out_hbm.at[pl.ds(base + c * _CHUNK, _CHUNK)])
        cp = nxt


def _gather_call(table, idx_flat):
    rows, d = idx_flat.shape[0], table.shape[1]
    mesh = plsc.VectorSubcoreMesh(core_axis_name="c", subcore_axis_name="s")
    k = functools.partial(
        pl.kernel,
        mesh=mesh,
        out_type=jax.ShapeDtypeStruct((rows, d), jnp.float32),
        scratch_types=[
            pltpu.VMEM((rows // 32,), jnp.int32),
            pltpu.VMEM((_CHUNK, d), jnp.float32),
            pltpu.VMEM((_CHUNK, d), jnp.float32),
            pltpu.SemaphoreType.DMA,
            pltpu.SemaphoreType.DMA,
        ],
    )(_gather_body)
    return k(table, idx_flat)


def _km_body(sel_ref, out_ref, *, L, D):
    BN = sel_ref.shape[0]
    B = BN // _NAGG
    NA, LD = _NAGG, L * D
    BK = B * _K
    x = sel_ref[...]
    x_hi = x.astype(jnp.bfloat16)
    x_lo = (x - x_hi.astype(jnp.float32)).astype(jnp.bfloat16)

    row_cl = lax.broadcasted_iota(jnp.int32, (BK, BN), 0)
    col_b = lax.broadcasted_iota(jnp.int32, (BK, BN), 1) // _NAGG
    penalty = jnp.where(row_cl // _K == col_b, 0.0, 1e9)

    def _dotT(a, b):
        return lax.dot_general(a, b, (((1,), (1,)), ((), ())),
                               preferred_element_type=jnp.float32)

    def labels_lbl(c):
        c2 = jnp.sum(c * c, axis=1, keepdims=True)
        c_hi = c.astype(jnp.bfloat16)
        c_lo = (c - c_hi.astype(jnp.float32)).astype(jnp.bfloat16)
        g = _dotT(c_hi, x_hi) + _dotT(c_hi, x_lo) + _dotT(c_lo, x_hi)
        d2 = c2 - 2.0 * g + penalty
        mn = jnp.min(d2, axis=0, keepdims=True)
        return jnp.min(jnp.where(d2 == mn, row_cl, BK), axis=0, keepdims=True)

    def labels_onehot(c):
        return (row_cl == labels_lbl(c)).astype(jnp.float32)

    def cond(carry):
        i, c, lbl_prev, changed = carry
        return jnp.logical_and(i < _ITERS, changed)

    def body(carry):
        i, c, lbl_prev, _ = carry
        lbl = labels_lbl(c)
        changed = jnp.any(lbl != lbl_prev)
        onehot = (row_cl == lbl).astype(jnp.float32)
        counts = jnp.sum(onehot, axis=1, keepdims=True)
        oh16 = onehot.astype(jnp.bfloat16)
        s = (lax.dot_general(oh16, x_hi, (((1,), (0,)), ((), ())),
                             preferred_element_type=jnp.float32)
             + lax.dot_general(oh16, x_lo, (((1,), (0,)), ((), ())),
                               preferred_element_type=jnp.float32))
        c = jnp.where(counts > 0.0, s / jnp.maximum(counts, 1.0), c)
        return (i + 1, c, lbl, changed)

    c0 = jnp.concatenate([x[b * NA:b * NA + _K, :] for b in range(B)], axis=0)
    lbl0 = jnp.full((1, BN), -1, dtype=jnp.int32)
    _, c, _, _ = lax.while_loop(cond, body, (0, c0, lbl0, True))

    onehot = labels_onehot(c)
    counts = jnp.sum(onehot, axis=1, keepdims=True)
    sum_x = x[:, 0:D]
    for l in range(1, L):
        sum_x = sum_x + x[:, l * D:(l + 1) * D]
    pooled = lax.dot_general(onehot, sum_x, (((1,), (0,)), ((), ())),
                             preferred_element_type=jnp.float32,
                             precision=lax.Precision.HIGHEST)
    centers = pooled / jnp.maximum(L * counts, 1.0)
    sel_b = (lax.broadcasted_iota(jnp.int32, (B, BK), 1) // _K
             == lax.broadcasted_iota(jnp.int32, (B, BK), 0)).astype(jnp.float32)
    ob = lax.dot_general(sel_b, centers, (((1,), (0,)), ((), ())),
                         preferred_element_type=jnp.float32,
                         precision=lax.Precision.HIGHEST) * (1.0 / _K)
    nrm = jnp.sqrt(jnp.sum(ob * ob, axis=1, keepdims=True))
    out_ref[...] = (ob / jnp.maximum(nrm, 1e-12))[None]


_KM_PROGS = 4


def _km_call(sel, L, D):
    BN, LD = sel.shape
    B = BN // _NAGG
    p = _KM_PROGS
    return pl.pallas_call(
        functools.partial(_km_body, L=L, D=D),
        grid=(p,),
        in_specs=[pl.BlockSpec((BN // p, LD), lambda i: (i, 0))],
        out_specs=pl.BlockSpec((1, B // p, D), lambda i: (i, 0, 0)),
        out_shape=jax.ShapeDtypeStruct((p, B // p, D), jnp.float32),
        compiler_params=pltpu.CompilerParams(
            vmem_limit_bytes=110 * 1024 * 1024),
    )(sel).reshape(B, D)


def kernel(patch_tokens, anomaly_maps):
    L, B, N, D = patch_tokens.shape
    a0 = anomaly_maps[..., 0]
    a1 = anomaly_maps[..., 1]
    idx_flat = _topk_call(a0, a1)
    table = patch_tokens.reshape(L * B * N, D)
    rows = _gather_call(table, idx_flat.reshape(B * _NAGG * L))
    sel = rows.reshape(B * _NAGG, L * D)
    return _km_call(sel, L, D)
